# baseline (speedup 1.0000x reference)
import functools

import jax
import jax.numpy as jnp
from jax import lax
from jax.experimental import pallas as pl
from jax.experimental.pallas import tpu as pltpu
from jax.experimental.pallas import tpu_sc as plsc

N = 50000
K = 27
C = 128
NW = 32
BC = 112
CHUNKS = 14
NP = NW * BC * CHUNKS
BT = 512
NT = NP // BT
ZROW = N
EPS = 1e-5


def _gemm_body(x_ref, w_ref, o_ref):
    o_ref[...] = jnp.dot(x_ref[...], w_ref[0],
                         preferred_element_type=jnp.float32)


def _gemm_bn_body(y_ref, s_ref, g_ref, b_ref, w_ref, o_ref):
    mean = s_ref[0:1, :] * (1.0 / N)
    var = s_ref[1:2, :] * (1.0 / N) - mean * mean
    inv = lax.rsqrt(var + EPS)
    x = (y_ref[...] - mean) * (inv * g_ref[0:1, :]) + b_ref[0:1, :]
    x = jnp.maximum(x, 0.0)
    t = pl.program_id(0)
    rows = t * BT + lax.broadcasted_iota(jnp.int32, (BT, 1), 0)
    x = jnp.where(rows < N, x, 0.0)
    o_ref[...] = jnp.dot(x, w_ref[0], preferred_element_type=jnp.float32)


def _stats_body(y_ref, o_ref):
    @pl.when(pl.program_id(0) == 0)
    def _init():
        o_ref[...] = jnp.zeros_like(o_ref)

    y = y_ref[...]
    o_ref[0:1, :] += jnp.sum(y, axis=0, keepdims=True)
    o_ref[1:2, :] += jnp.sum(y * y, axis=0, keepdims=True)


def _bn_relu_body(y_ref, s_ref, g_ref, b_ref, o_ref):
    mean = s_ref[0:1, :] * (1.0 / N)
    var = s_ref[1:2, :] * (1.0 / N) - mean * mean
    inv = lax.rsqrt(var + EPS)
    x = (y_ref[...] - mean) * (inv * g_ref[0:1, :]) + b_ref[0:1, :]
    o_ref[...] = jnp.maximum(x, 0.0)


def _gemm_all_k(x, w):
    return pl.pallas_call(
        _gemm_body,
        grid=(NT, K),
        in_specs=[pl.BlockSpec((BT, C), lambda t, k: (t, 0)),
                  pl.BlockSpec((1, C, C), lambda t, k: (k, 0, 0))],
        out_specs=pl.BlockSpec((BT, C), lambda t, k: (k * NT + t, 0)),
        out_shape=jax.ShapeDtypeStruct((K * NP, C), jnp.float32),
    )(x, w)


def _gemm_bn_all_k(y, s, g, b, w):
    return pl.pallas_call(
        _gemm_bn_body,
        grid=(NT, K),
        in_specs=[pl.BlockSpec((BT, C), lambda t, k: (t, 0)),
                  pl.BlockSpec((8, C), lambda t, k: (0, 0)),
                  pl.BlockSpec((1, C), lambda t, k: (0, 0)),
                  pl.BlockSpec((1, C), lambda t, k: (0, 0)),
                  pl.BlockSpec((1, C, C), lambda t, k: (k, 0, 0))],
        out_specs=pl.BlockSpec((BT, C), lambda t, k: (k * NT + t, 0)),
        out_shape=jax.ShapeDtypeStruct((K * NP, C), jnp.float32),
    )(y, s, g, b, w)


def _stats(y):
    return pl.pallas_call(
        _stats_body,
        grid=(NT,),
        in_specs=[pl.BlockSpec((BT, C), lambda t: (t, 0))],
        out_specs=pl.BlockSpec((8, C), lambda t: (0, 0)),
        out_shape=jax.ShapeDtypeStruct((8, C), jnp.float32),
    )(y)


def _bn_relu_final(y, s, g, b):
    return pl.pallas_call(
        _bn_relu_body,
        grid=(25,),
        in_specs=[pl.BlockSpec((2000, C), lambda t: (t, 0)),
                  pl.BlockSpec((8, C), lambda t: (0, 0)),
                  pl.BlockSpec((1, C), lambda t: (0, 0)),
                  pl.BlockSpec((1, C), lambda t: (0, 0))],
        out_specs=pl.BlockSpec((2000, C), lambda t: (t, 0)),
        out_shape=jax.ShapeDtypeStruct((N, C), jnp.float32),
    )(y, s, g, b)


def _sc_gather_body(z_hbm, idx_hbm, y_hbm, idx_v, acc_v, sem0, sem1):
    wid = lax.axis_index("s") * 2 + lax.axis_index("c")
    base = wid * (BC * CHUNKS)

    def chunk(ci, carry):
        cb = base + ci * BC
        pltpu.sync_copy(idx_hbm.at[wid * CHUNKS + ci], idx_v)
        pltpu.async_copy(z_hbm.at[idx_v.at[0]], acc_v, sem0).wait()
        cps = [pltpu.async_copy(z_hbm.at[idx_v.at[k]], acc_v, sem1, add=True)
               for k in range(1, K)]
        for cp in cps:
            cp.wait()
        pltpu.sync_copy(acc_v, y_hbm.at[pl.ds(cb, BC)])
        return carry

    lax.fori_loop(0, CHUNKS, chunk, 0)


def _sc_gather(z, idx_t):
    fn = pl.kernel(
        _sc_gather_body,
        out_type=jax.ShapeDtypeStruct((NP, C), jnp.float32),
        mesh=plsc.VectorSubcoreMesh(core_axis_name="c", subcore_axis_name="s"),
        scratch_types=[pltpu.VMEM((K, BC), jnp.int32),
                       pltpu.VMEM((BC, C), jnp.float32),
                       pltpu.SemaphoreType.DMA,
                       pltpu.SemaphoreType.DMA],
    )
    return fn(z, idx_t)


def kernel(features, nbr_idx, nbr_mask, W1, gamma1, beta1, W2, gamma2, beta2):
    x = jnp.pad(features, ((0, NP - N), (0, 0)))
    offs = (jnp.arange(K, dtype=jnp.int32) * NP)[None, :]
    fidx = jnp.where(nbr_mask, nbr_idx + offs, ZROW).astype(jnp.int32)
    fidx = jnp.pad(fidx, ((0, NP - N), (0, 0)), constant_values=ZROW)
    idx_t = fidx.reshape(NW * CHUNKS, BC, K).transpose(0, 2, 1)

    g1 = gamma1.reshape(1, C)
    b1 = beta1.reshape(1, C)
    g2 = gamma2.reshape(1, C)
    b2 = beta2.reshape(1, C)

    z1 = _gemm_all_k(x, W1)
    y1 = _sc_gather(z1, idx_t)
    s1 = _stats(y1)
    z2 = _gemm_bn_all_k(y1, s1, g1, b1, W2)
    y2 = _sc_gather(z2, idx_t)
    s2 = _stats(y2)
    return _bn_relu_final(y2, s2, g2, b2)
